# Initial kernel scaffold; baseline (speedup 1.0000x reference)
import jax
import jax.numpy as jnp
from jax.experimental import pallas as pl


def kernel(x, edge_index, batch, W0, b0, W1, b1, W2, b2, W3, b3):
    raise NotImplementedError("write your pallas kernel here")


# speedup vs baseline: 12.2500x; 12.2500x over previous
import functools

import jax
import jax.numpy as jnp
from jax import lax
from jax.experimental import pallas as pl
from jax.experimental.pallas import tpu as pltpu
from jax.experimental.pallas import tpu_sc as plsc

N = 10000
E = 320000
D_IN = 128
H = 64
C = 10
G = 64

NC = 2
NS = 16
NT = NC * NS
EPT = E // NT
K = 80
NCHUNK = EPT // K
RPT = 640
RLAST = N - 15 * RPT
NDEG = 10240
DPT = NDEG // NS

_mesh = plsc.VectorSubcoreMesh(core_axis_name="c", subcore_axis_name="s")
_f32 = jnp.float32


@functools.partial(
    pl.kernel,
    out_type=jax.ShapeDtypeStruct((NC, 1, NDEG), _f32),
    mesh=_mesh,
    scratch_types=[
        pltpu.VMEM((K,), jnp.int32),
        pltpu.VMEM((K,), _f32),
        pltpu.VMEM_SHARED((NDEG,), _f32),
    ],
)
def _deg_sc(dst_hbm, ones_hbm, zeros_hbm, out_hbm, idx_d, ones_v, dacc):
    c = lax.axis_index("c")
    s = lax.axis_index("s")
    tid = c * NS + s
    pltpu.sync_copy(zeros_hbm, dacc.at[pl.ds(s * DPT, DPT)])
    pltpu.sync_copy(ones_hbm, ones_v)
    plsc.subcore_barrier()

    def body(i, carry):
        base = tid * EPT + i * K
        pltpu.sync_copy(dst_hbm.at[pl.ds(base, K)], idx_d)
        pltpu.sync_copy(ones_v, dacc.at[idx_d], add=True)
        return carry

    lax.fori_loop(0, NCHUNK, body, 0)
    plsc.subcore_barrier()
    pltpu.sync_copy(dacc.at[pl.ds(s * DPT, DPT)],
                    out_hbm.at[c, 0, pl.ds(s * DPT, DPT)])


@functools.partial(
    pl.kernel,
    out_type=jax.ShapeDtypeStruct((NC, N, H), _f32),
    mesh=_mesh,
    scratch_types=[
        pltpu.VMEM((K,), jnp.int32),
        pltpu.VMEM((K,), jnp.int32),
        pltpu.VMEM((K, H), _f32),
        pltpu.VMEM_SHARED((N, H), _f32),
        pltpu.SemaphoreType.DMA,
    ],
    compiler_params=pltpu.CompilerParams(use_tc_tiling_on_sc=False),
)
def _agg_sc(s_hbm, src_hbm, dst_hbm, zrows_hbm, out_hbm,
            idx_s, idx_d, rows, acc, sem):
    c = lax.axis_index("c")
    s = lax.axis_index("s")
    tid = c * NS + s

    @pl.when(s < NS - 1)
    def _():
        pltpu.sync_copy(zrows_hbm, acc.at[pl.ds(s * RPT, RPT)])

    @pl.when(s == NS - 1)
    def _():
        pltpu.sync_copy(zrows_hbm.at[pl.ds(0, RLAST)],
                        acc.at[pl.ds((NS - 1) * RPT, RLAST)])

    plsc.subcore_barrier()

    def body(i, carry):
        base = tid * EPT + i * K
        pltpu.sync_copy(src_hbm.at[pl.ds(base, K)], idx_s)
        pltpu.sync_copy(dst_hbm.at[pl.ds(base, K)], idx_d)
        pltpu.async_copy(s_hbm.at[idx_s], rows, sem).wait()
        pltpu.sync_copy(rows, acc.at[idx_d], add=True)
        return carry

    lax.fori_loop(0, NCHUNK, body, 0)
    plsc.subcore_barrier()

    @pl.when(s < NS - 1)
    def _():
        pltpu.sync_copy(acc.at[pl.ds(s * RPT, RPT)],
                        out_hbm.at[c, pl.ds(s * RPT, RPT)])

    @pl.when(s == NS - 1)
    def _():
        pltpu.sync_copy(acc.at[pl.ds((NS - 1) * RPT, RLAST)],
                        out_hbm.at[c, pl.ds((NS - 1) * RPT, RLAST)])


def _prep_body(degcol_ref, x_ref, w0_ref, s0_ref, dinv_ref):
    dinv = lax.rsqrt(degcol_ref[...])
    dinv_ref[...] = dinv
    s0_ref[...] = jnp.dot(x_ref[...], w0_ref[...],
                          preferred_element_type=_f32) * dinv


_prep_tc = pl.pallas_call(
    _prep_body,
    out_shape=(jax.ShapeDtypeStruct((N, H), _f32),
               jax.ShapeDtypeStruct((N, 1), _f32)),
)


def _mid_body(p_ref, sprev_ref, dinv_ref, b_ref, w_ref, out_ref):
    dinv = dinv_ref[...]
    t = dinv * (p_ref[0] + p_ref[1] + sprev_ref[...]) + b_ref[...]
    xl = jnp.maximum(t, 0.0)
    out_ref[...] = jnp.dot(xl, w_ref[...], preferred_element_type=_f32) * dinv


_mid_tc = pl.pallas_call(
    _mid_body,
    out_shape=jax.ShapeDtypeStruct((N, H), _f32),
)


def _mid3_body(p_ref, sprev_ref, dinv_ref, b_ref, out_ref):
    dinv = dinv_ref[...]
    t = dinv * (p_ref[0] + p_ref[1] + sprev_ref[...]) + b_ref[...]
    out_ref[...] = jnp.maximum(t, 0.0) * dinv


_mid3_tc = pl.pallas_call(
    _mid3_body,
    out_shape=jax.ShapeDtypeStruct((N, H), _f32),
)


def _final_body(p_ref, sprev_ref, dinv_ref, batch_ref, w3_ref, b3_ref,
                out_ref):
    t = dinv_ref[...] * (p_ref[0] + p_ref[1] + sprev_ref[...])
    gid = lax.broadcasted_iota(jnp.int32, (G, N), 0)
    oh = (batch_ref[...] == gid).astype(_f32)
    sums = jnp.dot(oh, t, preferred_element_type=_f32)
    cnt = jnp.sum(oh, axis=1, keepdims=True)
    num = jnp.dot(sums, w3_ref[...], preferred_element_type=_f32)
    num = num + cnt * b3_ref[...]
    out_ref[...] = num / jnp.maximum(cnt, 1.0)


_final_tc = pl.pallas_call(
    _final_body,
    out_shape=jax.ShapeDtypeStruct((G, C), _f32),
)


def kernel(x, edge_index, batch, W0, b0, W1, b1, W2, b2, W3, b3):
    src = edge_index[0].astype(jnp.int32)
    dst = edge_index[1].astype(jnp.int32)
    batch2 = batch.reshape(1, N).astype(jnp.int32)
    zrows = jnp.zeros((RPT, H), _f32)
    zeros1 = jnp.zeros((DPT,), _f32)
    ones_k = jnp.ones((K,), _f32)

    degp = _deg_sc(dst, ones_k, zeros1)
    degcol = (degp[0, 0, :N] + degp[1, 0, :N] + 1.0).reshape(N, 1)
    s0, dinv = _prep_tc(degcol, x, W0)

    p = _agg_sc(s0, src, dst, zrows)
    s1 = _mid_tc(p, s0, dinv, b0.reshape(1, H), W1)
    p = _agg_sc(s1, src, dst, zrows)
    s2 = _mid_tc(p, s1, dinv, b1.reshape(1, H), W2)
    p = _agg_sc(s2, src, dst, zrows)
    s3 = _mid3_tc(p, s2, dinv, b2.reshape(1, H))
    p = _agg_sc(s3, src, dst, zrows)
    return _final_tc(p, s3, dinv, batch2, W3, b3.reshape(1, C))


# speedup vs baseline: 23.0491x; 1.8816x over previous
import functools

import jax
import jax.numpy as jnp
from jax import lax
from jax.experimental import pallas as pl
from jax.experimental.pallas import tpu as pltpu
from jax.experimental.pallas import tpu_sc as plsc

N = 10000
E = 320000
D_IN = 128
H = 64
C = 10
G = 64

NC = 2
NS = 16
NT = NC * NS
EPT = E // NT
K = 80
NCHUNK = EPT // K
RPT = 640
RLAST = N - 15 * RPT
NDEG = 10240
DPT = NDEG // NS

_mesh = plsc.VectorSubcoreMesh(core_axis_name="c", subcore_axis_name="s")
_f32 = jnp.float32


@functools.partial(
    pl.kernel,
    out_type=jax.ShapeDtypeStruct((NC, 1, NDEG), _f32),
    mesh=_mesh,
    scratch_types=[
        pltpu.VMEM((K,), jnp.int32),
        pltpu.VMEM((K,), _f32),
        pltpu.VMEM_SHARED((NDEG,), _f32),
    ],
)
def _deg_sc(dst_hbm, ones_hbm, zeros_hbm, out_hbm, idx_d, ones_v, dacc):
    c = lax.axis_index("c")
    s = lax.axis_index("s")
    tid = c * NS + s
    pltpu.sync_copy(zeros_hbm, dacc.at[pl.ds(s * DPT, DPT)])
    pltpu.sync_copy(ones_hbm, ones_v)
    plsc.subcore_barrier()

    def body(i, carry):
        base = tid * EPT + i * K
        pltpu.sync_copy(dst_hbm.at[pl.ds(base, K)], idx_d)
        pltpu.sync_copy(ones_v, dacc.at[idx_d], add=True)
        return carry

    lax.fori_loop(0, NCHUNK, body, 0)
    plsc.subcore_barrier()
    pltpu.sync_copy(dacc.at[pl.ds(s * DPT, DPT)],
                    out_hbm.at[c, 0, pl.ds(s * DPT, DPT)])


@functools.partial(
    pl.kernel,
    out_type=jax.ShapeDtypeStruct((NC, N, H), _f32),
    mesh=_mesh,
    scratch_types=[
        pltpu.VMEM((NCHUNK, K), jnp.int32),
        pltpu.VMEM((K,), jnp.int32),
        pltpu.VMEM((K,), jnp.int32),
        pltpu.VMEM((K, H), _f32),
        pltpu.VMEM((K, H), _f32),
        pltpu.VMEM_SHARED((N, H), _f32),
        pltpu.SemaphoreType.DMA,
        pltpu.SemaphoreType.DMA,
    ],
    compiler_params=pltpu.CompilerParams(use_tc_tiling_on_sc=False),
)
def _agg_sc(s_hbm, src_hbm, dst_hbm, zrows_hbm, out_hbm,
            srcbuf, dstb0, dstb1, rows0, rows1, acc, sem0, sem1):
    c = lax.axis_index("c")
    s = lax.axis_index("s")
    tid = c * NS + s

    pltpu.sync_copy(src_hbm.at[tid], srcbuf)
    pltpu.sync_copy(dst_hbm.at[tid, 0], dstb0)
    pltpu.async_copy(s_hbm.at[srcbuf.at[0]], rows0, sem0)
    pltpu.sync_copy(dst_hbm.at[tid, 1], dstb1)
    pltpu.async_copy(s_hbm.at[srcbuf.at[1]], rows1, sem1)

    @pl.when(s < NS - 1)
    def _():
        pltpu.sync_copy(zrows_hbm, acc.at[pl.ds(s * RPT, RPT)])

    @pl.when(s == NS - 1)
    def _():
        pltpu.sync_copy(zrows_hbm.at[pl.ds(0, RLAST)],
                        acc.at[pl.ds((NS - 1) * RPT, RLAST)])

    plsc.subcore_barrier()

    def _step(i, dstb, rows, sem):
        pltpu.make_async_copy(s_hbm.at[srcbuf.at[i]], rows, sem).wait()
        pltpu.sync_copy(rows, acc.at[dstb], add=True)

        @pl.when(i + 2 < NCHUNK)
        def _():
            pltpu.sync_copy(dst_hbm.at[tid, i + 2], dstb)
            pltpu.async_copy(s_hbm.at[srcbuf.at[i + 2]], rows, sem)

    def body(i, carry):
        @pl.when(lax.rem(i, 2) == 0)
        def _():
            _step(i, dstb0, rows0, sem0)

        @pl.when(lax.rem(i, 2) == 1)
        def _():
            _step(i, dstb1, rows1, sem1)

        return carry

    lax.fori_loop(0, NCHUNK, body, 0)
    plsc.subcore_barrier()

    @pl.when(s < NS - 1)
    def _():
        pltpu.sync_copy(acc.at[pl.ds(s * RPT, RPT)],
                        out_hbm.at[c, pl.ds(s * RPT, RPT)])

    @pl.when(s == NS - 1)
    def _():
        pltpu.sync_copy(acc.at[pl.ds((NS - 1) * RPT, RLAST)],
                        out_hbm.at[c, pl.ds((NS - 1) * RPT, RLAST)])


def _prep_body(degcol_ref, x_ref, w0_ref, s0_ref, dinv_ref):
    dinv = lax.rsqrt(degcol_ref[...])
    dinv_ref[...] = dinv
    s0_ref[...] = jnp.dot(x_ref[...], w0_ref[...],
                          preferred_element_type=_f32) * dinv


_prep_tc = pl.pallas_call(
    _prep_body,
    out_shape=(jax.ShapeDtypeStruct((N, H), _f32),
               jax.ShapeDtypeStruct((N, 1), _f32)),
)


def _mid_body(p_ref, sprev_ref, dinv_ref, b_ref, w_ref, out_ref):
    dinv = dinv_ref[...]
    t = dinv * (p_ref[0] + p_ref[1] + sprev_ref[...]) + b_ref[...]
    xl = jnp.maximum(t, 0.0)
    out_ref[...] = jnp.dot(xl, w_ref[...], preferred_element_type=_f32) * dinv


_mid_tc = pl.pallas_call(
    _mid_body,
    out_shape=jax.ShapeDtypeStruct((N, H), _f32),
)


def _mid3_body(p_ref, sprev_ref, dinv_ref, b_ref, out_ref):
    dinv = dinv_ref[...]
    t = dinv * (p_ref[0] + p_ref[1] + sprev_ref[...]) + b_ref[...]
    out_ref[...] = jnp.maximum(t, 0.0) * dinv


_mid3_tc = pl.pallas_call(
    _mid3_body,
    out_shape=jax.ShapeDtypeStruct((N, H), _f32),
)


def _final_body(p_ref, sprev_ref, dinv_ref, batch_ref, w3_ref, b3_ref,
                out_ref):
    t = dinv_ref[...] * (p_ref[0] + p_ref[1] + sprev_ref[...])
    gid = lax.broadcasted_iota(jnp.int32, (G, N), 0)
    oh = (batch_ref[...] == gid).astype(_f32)
    sums = jnp.dot(oh, t, preferred_element_type=_f32)
    cnt = jnp.sum(oh, axis=1, keepdims=True)
    num = jnp.dot(sums, w3_ref[...], preferred_element_type=_f32)
    num = num + cnt * b3_ref[...]
    out_ref[...] = num / jnp.maximum(cnt, 1.0)


_final_tc = pl.pallas_call(
    _final_body,
    out_shape=jax.ShapeDtypeStruct((G, C), _f32),
)


def kernel(x, edge_index, batch, W0, b0, W1, b1, W2, b2, W3, b3):
    src = edge_index[0].astype(jnp.int32)
    dst = edge_index[1].astype(jnp.int32)
    src3 = src.reshape(NT, NCHUNK, K)
    dst3 = dst.reshape(NT, NCHUNK, K)
    batch2 = batch.reshape(1, N).astype(jnp.int32)
    zrows = jnp.zeros((RPT, H), _f32)
    zeros1 = jnp.zeros((DPT,), _f32)
    ones_k = jnp.ones((K,), _f32)

    degp = _deg_sc(dst, ones_k, zeros1)
    degcol = (degp[0, 0, :N] + degp[1, 0, :N] + 1.0).reshape(N, 1)
    s0, dinv = _prep_tc(degcol, x, W0)

    p = _agg_sc(s0, src3, dst3, zrows)
    s1 = _mid_tc(p, s0, dinv, b0.reshape(1, H), W1)
    p = _agg_sc(s1, src3, dst3, zrows)
    s2 = _mid_tc(p, s1, dinv, b1.reshape(1, H), W2)
    p = _agg_sc(s2, src3, dst3, zrows)
    s3 = _mid3_tc(p, s2, dinv, b2.reshape(1, H))
    p = _agg_sc(s3, src3, dst3, zrows)
    return _final_tc(p, s3, dinv, batch2, W3, b3.reshape(1, C))
